# baseline (speedup 1.0000x reference)
import functools

import jax
import jax.numpy as jnp
from jax import lax
from jax.experimental import pallas as pl
from jax.experimental.pallas import tpu as pltpu
from jax.experimental.pallas import tpu_sc as plsc


def _make_gather(B, V, D):
    info = plsc.get_sparse_core_info()
    nc, ns, L = info.num_cores, info.num_subcores, info.num_lanes
    nw = nc * ns
    assert D % nw == 0
    d_per_w = D // nw
    S = 49920
    assert S % 128 == 0
    chunk = 4096
    assert B % chunk == 0
    n_chunks = B // chunk
    mesh = plsc.VectorSubcoreMesh(core_axis_name="c", subcore_axis_name="s")

    @functools.partial(
        pl.kernel,
        mesh=mesh,
        out_type=jax.ShapeDtypeStruct((D, B), jnp.float32),
        scratch_types=[
            pltpu.VMEM((S,), jnp.float32),
            pltpu.VMEM((V - S,), jnp.float32),
            pltpu.VMEM((B,), jnp.float32),
            pltpu.VMEM((chunk,), jnp.int32),
            pltpu.VMEM((chunk,), jnp.int32),
            pltpu.SemaphoreType.DMA,
            pltpu.SemaphoreType.DMA,
            pltpu.SemaphoreType.DMA,
            pltpu.SemaphoreType.DMA,
        ],
        compiler_params=pltpu.CompilerParams(needs_layout_passes=False),
    )
    def gather_kernel(
        idx_hbm, tt_hbm, out_hbm, va, vb, stage, idx_a, idx_b,
        sem_a, sem_b, sem_i, sem_o,
    ):
        wid = lax.axis_index("s") * nc + lax.axis_index("c")
        idx_bufs = [idx_a, idx_b]

        def load_half(d, half):
            row = tt_hbm.at[d]
            if half == 0:
                return pltpu.make_async_copy(row.at[pl.ds(0, S)], va, sem_a)
            return pltpu.make_async_copy(row.at[pl.ds(S, V - S)], vb, sem_b)

        def idx_load(ci):
            cp = pltpu.make_async_copy(
                idx_hbm.at[pl.ds(ci * chunk, chunk)], idx_bufs[ci % 2], sem_i
            )
            cp.start()
            return cp

        d0 = wid * d_per_w
        cp_va = load_half(d0, 0)
        cp_va.start()
        cp_vb = load_half(d0, 1)
        cp_vb.start()

        out_cps = []
        idx_cp = idx_load(0)
        cp_va.wait()
        for di in range(d_per_w):
            d = d0 + di
            for ci in range(n_chunks):
                idx_cp.wait()
                if ci + 1 < n_chunks:
                    idx_cp = idx_load(ci + 1)
                idx_v = idx_bufs[ci % 2]

                @plsc.parallel_loop(0, chunk, L, unroll=8)
                def _(i):
                    ids16 = idx_v[pl.ds(i, L)]
                    ids_a = jnp.minimum(ids16, S - 1)
                    stage[pl.ds(ci * chunk + i, L)] = plsc.load_gather(va, [ids_a])

            cp_vb.wait()
            if di + 1 < d_per_w:
                cp_va = load_half(d + 1, 0)
                cp_va.start()

            idx_cp = idx_load(0)
            for ci in range(n_chunks):
                idx_cp.wait()
                if ci + 1 < n_chunks:
                    idx_cp = idx_load(ci + 1)
                idx_v = idx_bufs[ci % 2]

                @plsc.parallel_loop(0, chunk, L, unroll=8)
                def _(i):
                    ids16 = idx_v[pl.ds(i, L)]
                    hi = ids16 >= S
                    ids_b = jnp.maximum(ids16, S) - S
                    g_b = plsc.load_gather(vb, [ids_b])
                    pos = ci * chunk + i + lax.iota(jnp.int32, L)
                    plsc.store_scatter(stage, [pos], g_b, mask=hi)

            if di + 1 < d_per_w:
                cp_vb = load_half(d + 1, 1)
                cp_vb.start()

            cp_o = pltpu.make_async_copy(stage, out_hbm.at[d], sem_o)
            cp_o.start()
            out_cps.append(cp_o)
            if di + 1 < d_per_w:
                out_cps.pop(0).wait()
                idx_cp = idx_load(0)
                cp_va.wait()
        for cp in out_cps:
            cp.wait()

    return gather_kernel


def kernel(target_gene_ids, target_embedding):
    ids = target_gene_ids
    if ids.ndim > 1:
        ids = jnp.squeeze(ids)
    (B,) = ids.shape
    V, D = target_embedding.shape
    fn = _make_gather(B, V, D)
    out_t = fn(ids.astype(jnp.int32), target_embedding.T)
    return out_t.T


# speedup vs baseline: 1.3906x; 1.3906x over previous
import functools

import jax
import jax.numpy as jnp
from jax import lax
from jax.experimental import pallas as pl
from jax.experimental.pallas import tpu as pltpu
from jax.experimental.pallas import tpu_sc as plsc


def _make_gather(B, V, D):
    info = plsc.get_sparse_core_info()
    nc, ns, L = info.num_cores, info.num_subcores, info.num_lanes
    nw = nc * ns
    assert D % nw == 0
    d_per_w = D // nw
    chunk = 4096
    assert B % chunk == 0
    n_chunks = B // chunk
    mesh = plsc.VectorSubcoreMesh(core_axis_name="c", subcore_axis_name="s")

    @functools.partial(
        pl.kernel,
        mesh=mesh,
        out_type=jax.ShapeDtypeStruct((D, B), jnp.float32),
        scratch_types=[
            pltpu.VMEM((V,), jnp.float32),
            pltpu.VMEM((B,), jnp.int32),
            pltpu.VMEM((chunk,), jnp.float32),
            pltpu.VMEM((chunk,), jnp.float32),
            pltpu.SemaphoreType.DMA,
            pltpu.SemaphoreType.DMA,
        ],
        compiler_params=pltpu.CompilerParams(needs_layout_passes=False),
    )
    def gather_kernel(idx_hbm, tt_hbm, out_hbm, vec_v, idx_v, out_a, out_b, sem, osem):
        wid = lax.axis_index("s") * nc + lax.axis_index("c")
        out_bufs = [out_a, out_b]

        first_vec = pltpu.make_async_copy(tt_hbm.at[wid * d_per_w], vec_v, sem)
        first_vec.start()
        pltpu.sync_copy(idx_hbm, idx_v)
        first_vec.wait()

        out_cps = [None, None]
        for di in range(d_per_w):
            d = wid * d_per_w + di
            if di > 0:
                pltpu.sync_copy(tt_hbm.at[d], vec_v)
            for ci in range(n_chunks):
                buf = ci % 2
                out_v = out_bufs[buf]
                if out_cps[buf] is not None:
                    out_cps[buf].wait()
                    out_cps[buf] = None

                @plsc.parallel_loop(0, chunk, L, unroll=8)
                def _(i):
                    ids16 = idx_v[pl.ds(ci * chunk + i, L)]
                    out_v[pl.ds(i, L)] = plsc.load_gather(vec_v, [ids16])

                cp = pltpu.make_async_copy(
                    out_v, out_hbm.at[d, pl.ds(ci * chunk, chunk)], osem
                )
                cp.start()
                out_cps[buf] = cp
        for cp in out_cps:
            if cp is not None:
                cp.wait()

    return gather_kernel


def kernel(target_gene_ids, target_embedding):
    ids = target_gene_ids
    if ids.ndim > 1:
        ids = jnp.squeeze(ids)
    (B,) = ids.shape
    V, D = target_embedding.shape
    fn = _make_gather(B, V, D)
    out_t = fn(ids.astype(jnp.int32), target_embedding.T)
    return out_t.T
